# Initial kernel scaffold; baseline (speedup 1.0000x reference)
import jax
import jax.numpy as jnp
from jax.experimental import pallas as pl


def kernel(embeds, centroids_init):
    raise NotImplementedError("write your pallas kernel here")


# speedup vs baseline: 11.4138x; 11.4138x over previous
import jax
import jax.numpy as jnp
from jax.experimental import pallas as pl

_N = 4096
_D = 32
_K = 512
_ITERS = 1000
_DP = 40


def _kmeans_body(embeds_ref, eaug_t_ref, ct_ref, cents_ref, idx_ref, nums_ref):
    embeds = embeds_ref[...]
    eaug_t = eaug_t_ref[...]
    x_sq = jnp.sum(embeds * embeds, axis=1, keepdims=True)
    iota_k = jax.lax.broadcasted_iota(jnp.int32, (_N, _K), 1)

    def step(c_t):
        c_sq = jnp.sum(c_t * c_t, axis=0, keepdims=True)
        prod = jax.lax.dot_general(
            embeds, c_t, (((1,), (0,)), ((), ())),
            preferred_element_type=jnp.float32,
            precision=jax.lax.Precision.DEFAULT)
        dists = x_sq - 2.0 * prod + c_sq
        dmin = jnp.min(dists, axis=1, keepdims=True)
        idx2d = jnp.min(jnp.where(dists == dmin, iota_k, _K),
                        axis=1, keepdims=True)
        onehot = (iota_k == idx2d).astype(jnp.float32)
        acc_t = jax.lax.dot_general(
            eaug_t, onehot, (((1,), (0,)), ((), ())),
            preferred_element_type=jnp.float32,
            precision=jax.lax.Precision.HIGHEST)
        counts = acc_t[_D:_D + 1, :]
        new_ct = acc_t[:_D, :] / (counts + 1e-6)
        return new_ct, idx2d, counts

    def body(i, carry):
        return step(carry[0])

    init = (ct_ref[...],
            jnp.zeros((_N, 1), jnp.int32),
            jnp.zeros((1, _K), jnp.float32))
    c_t, idx2d, counts = jax.lax.fori_loop(0, _ITERS, body, init)
    cents_ref[...] = c_t.T
    idx_ref[...] = idx2d
    nums_ref[...] = counts.T


def kernel(embeds, centroids_init):
    eaug_t = jnp.zeros((_DP, _N), jnp.float32)
    eaug_t = eaug_t.at[:_D, :].set(embeds.T)
    eaug_t = eaug_t.at[_D, :].set(1.0)
    cents, idx2d, nums = pl.pallas_call(
        _kmeans_body,
        out_shape=(
            jax.ShapeDtypeStruct((_K, _D), jnp.float32),
            jax.ShapeDtypeStruct((_N, 1), jnp.int32),
            jax.ShapeDtypeStruct((_K, 1), jnp.float32),
        ),
    )(embeds, eaug_t, centroids_init.T)
    return cents, idx2d[:, 0], nums


# speedup vs baseline: 12.0333x; 1.0543x over previous
import jax
import jax.numpy as jnp
from jax.experimental import pallas as pl

_N = 4096
_D = 32
_K = 512
_ITERS = 1000
_DP = 40


def _kmeans_body(embeds_ref, esplit_ref, ct_ref, cents_ref, idx_ref, nums_ref):
    embeds = embeds_ref[...]
    esplit = esplit_ref[...]
    x_sq = jnp.sum(embeds * embeds, axis=1, keepdims=True)
    x_sq_b = jnp.broadcast_to(x_sq, (_N, _K))
    iota_f = jax.lax.broadcasted_iota(jnp.int32, (_N, _K), 1).astype(jnp.float32)

    def step(c_t):
        c_sq = jnp.sum(c_t * c_t, axis=0, keepdims=True)
        prod = jax.lax.dot_general(
            embeds, c_t, (((1,), (0,)), ((), ())),
            preferred_element_type=jnp.float32,
            precision=jax.lax.Precision.DEFAULT)
        dists = x_sq_b - 2.0 * prod + c_sq
        dmin = jnp.min(dists, axis=1, keepdims=True)
        idxf = jnp.min(jnp.where(dists == dmin, iota_f, jnp.float32(_K)),
                       axis=1, keepdims=True)
        onehot = jnp.where(iota_f == idxf, jnp.float32(1),
                           jnp.float32(0))
        acc_t = jax.lax.dot_general(
            esplit, onehot, (((1,), (0,)), ((), ())),
            preferred_element_type=jnp.float32,
            precision=jax.lax.Precision.HIGHEST)
        counts = acc_t[_D:_D + 1, :]
        new_ct = acc_t[:_D, :] / (counts + 1e-6)
        return new_ct, idxf, counts

    def body(i, carry):
        return step(carry[0])

    init = (ct_ref[...],
            jnp.zeros((_N, 1), jnp.float32),
            jnp.zeros((1, _K), jnp.float32))
    c_t, idxf, counts = jax.lax.fori_loop(0, _ITERS, body, init)
    cents_ref[...] = c_t.T
    idx_ref[...] = idxf.astype(jnp.int32)
    nums_ref[...] = counts.T


def kernel(embeds, centroids_init):
    eaug_t = jnp.zeros((_DP, _N), jnp.float32)
    eaug_t = eaug_t.at[:_D, :].set(embeds.T)
    eaug_t = eaug_t.at[_D, :].set(1.0)
    esplit = eaug_t
    cents, idx2d, nums = pl.pallas_call(
        _kmeans_body,
        out_shape=(
            jax.ShapeDtypeStruct((_K, _D), jnp.float32),
            jax.ShapeDtypeStruct((_N, 1), jnp.int32),
            jax.ShapeDtypeStruct((_K, 1), jnp.float32),
        ),
    )(embeds, esplit, centroids_init.T)
    return cents, idx2d[:, 0], nums


# speedup vs baseline: 684.0226x; 56.8442x over previous
import jax
import jax.numpy as jnp
from jax.experimental import pallas as pl

_N = 4096
_D = 32
_K = 512
_ITERS = 1000
_DP = 40


def _kmeans_body(embeds_ref, esplit_ref, ct_ref, cents_ref, idx_ref, nums_ref):
    embeds = embeds_ref[...]
    esplit = esplit_ref[...]
    x_sq = jnp.sum(embeds * embeds, axis=1, keepdims=True)
    x_sq_b = jnp.broadcast_to(x_sq, (_N, _K))
    iota_f = jax.lax.broadcasted_iota(jnp.int32, (_N, _K), 1).astype(jnp.float32)

    def step(c_t):
        c_sq = jnp.sum(c_t * c_t, axis=0, keepdims=True)
        prod = jax.lax.dot_general(
            embeds, c_t, (((1,), (0,)), ((), ())),
            preferred_element_type=jnp.float32,
            precision=jax.lax.Precision.DEFAULT)
        dists = x_sq_b - 2.0 * prod + c_sq
        dmin = jnp.min(dists, axis=1, keepdims=True)
        idxf = jnp.min(jnp.where(dists == dmin, iota_f, jnp.float32(_K)),
                       axis=1, keepdims=True)
        onehot = jnp.where(iota_f == idxf, jnp.float32(1),
                           jnp.float32(0))
        acc_t = jax.lax.dot_general(
            esplit, onehot, (((1,), (0,)), ((), ())),
            preferred_element_type=jnp.float32,
            precision=jax.lax.Precision.HIGHEST)
        counts = acc_t[_D:_D + 1, :]
        new_ct = acc_t[:_D, :] / (counts + 1e-6)
        return new_ct, idxf, counts

    def cond(carry):
        i, _, _, _, same = carry
        return jnp.logical_and(i < _ITERS, jnp.logical_not(same))

    def body(carry):
        i, c_t, _, _, _ = carry
        new_ct, idxf, counts = step(c_t)
        same = jnp.all(new_ct == c_t)
        return (i + 1, new_ct, idxf, counts, same)

    init = (jnp.int32(0), ct_ref[...],
            jnp.zeros((_N, 1), jnp.float32),
            jnp.zeros((1, _K), jnp.float32),
            jnp.bool_(False))
    _, c_t, idxf, counts, _ = jax.lax.while_loop(cond, body, init)
    cents_ref[...] = c_t.T
    idx_ref[...] = idxf.astype(jnp.int32)
    nums_ref[...] = counts.T


def kernel(embeds, centroids_init):
    eaug_t = jnp.zeros((_DP, _N), jnp.float32)
    eaug_t = eaug_t.at[:_D, :].set(embeds.T)
    eaug_t = eaug_t.at[_D, :].set(1.0)
    esplit = eaug_t
    cents, idx2d, nums = pl.pallas_call(
        _kmeans_body,
        out_shape=(
            jax.ShapeDtypeStruct((_K, _D), jnp.float32),
            jax.ShapeDtypeStruct((_N, 1), jnp.int32),
            jax.ShapeDtypeStruct((_K, 1), jnp.float32),
        ),
    )(embeds, esplit, centroids_init.T)
    return cents, idx2d[:, 0], nums
